# baseline (speedup 1.0000x reference)
import functools

import jax
import jax.numpy as jnp
import numpy as np
from jax import lax
from jax.experimental import pallas as pl
from jax.experimental.pallas import tpu as pltpu, tpu_sc as plsc

N_V = 10000
N_V_PAD = 10240
E = 320000
D = 16
CHUNK = 128
N_CHUNKS = E // CHUNK
NC, NS = 2, 16
NW = NC * NS
CPT = N_CHUNKS // NW
LEFT = N_CHUNKS - CPT * NW
ROWS_PER_TILE = N_V_PAD // NS

_COL_TABLE = (np.arange(D, dtype=np.int32)[None, :] * CHUNK
              + np.arange(CHUNK, dtype=np.int32)[:, None])

_mesh = plsc.VectorSubcoreMesh(core_axis_name="c", subcore_axis_name="s")


@functools.partial(
    pl.kernel,
    out_type=jax.ShapeDtypeStruct((NC, N_V_PAD, D), jnp.float32),
    mesh=_mesh,
    compiler_params=pltpu.CompilerParams(
        use_tc_tiling_on_sc=False, needs_layout_passes=False),
    scratch_types=[
        pltpu.VMEM((CPT + 1, CHUNK), jnp.int32),
        pltpu.VMEM((CHUNK, D), jnp.int32),
        pltpu.VMEM((2, D * CHUNK), jnp.float32),
        pltpu.VMEM((2, CHUNK, D), jnp.float32),
        pltpu.VMEM((ROWS_PER_TILE, D), jnp.float32),
        pltpu.VMEM_SHARED((N_V_PAD, D), jnp.float32),
        pltpu.SemaphoreType.DMA,
        pltpu.SemaphoreType.DMA,
        pltpu.SemaphoreType.DMA,
        pltpu.SemaphoreType.DMA,
    ],
)
def _scatter_sc(idx_hbm, edge_hbm, col_hbm, out_hbm, idx_v, col_v, stage_v,
                trans_v, zbuf, acc, sem_f0, sem_f1, sem_s0, sem_s1):
    c = lax.axis_index("c")
    s = lax.axis_index("s")
    wid = c * NS + s
    row_base = pl.multiple_of(s * ROWS_PER_TILE, ROWS_PER_TILE)
    sem_f = (sem_f0, sem_f1)
    sem_s = (sem_s0, sem_s1)
    chunk0 = wid * CPT

    zero = jnp.zeros((D,), jnp.float32)

    def _zrow(i, carry):
        for r in range(8):
            zbuf[i * 8 + r, :] = zero
        return carry

    lax.fori_loop(0, ROWS_PER_TILE // 8, _zrow, 0)
    pltpu.sync_copy(zbuf, acc.at[pl.ds(row_base, ROWS_PER_TILE)])

    pltpu.sync_copy(col_hbm, col_v)
    pltpu.sync_copy(idx_hbm.at[pl.ds(chunk0, CPT), 1, :],
                    idx_v.at[pl.ds(0, CPT)])

    @pl.when(wid < LEFT)
    def _():
        pltpu.sync_copy(idx_hbm.at[NW * CPT + wid, 1, :], idx_v.at[CPT])

    plsc.subcore_barrier()

    HALF = D * CHUNK // 2

    def _fill(ec, b):
        pltpu.async_copy(edge_hbm.at[0, ec],
                         stage_v.at[b, pl.ds(0, HALF)], sem_f[b])
        pltpu.async_copy(edge_hbm.at[1, ec],
                         stage_v.at[b, pl.ds(HALF, HALF)], sem_f[b])

    def _wait_fill(b):
        pltpu.make_async_copy(
            edge_hbm.at[0, 0], stage_v.at[b, pl.ds(0, HALF)],
            sem_f[b]).wait()
        pltpu.make_async_copy(
            edge_hbm.at[1, 0], stage_v.at[b, pl.ds(HALF, HALF)],
            sem_f[b]).wait()

    def _wait_scat(b):
        pltpu.make_async_copy(
            trans_v.at[b], acc.at[idx_v.at[0]], sem_s[b]).wait()

    def _transpose(b):
        stage_b = stage_v.at[b]

        @plsc.parallel_loop(0, CHUNK, unroll=8)
        def _trow(e):
            v = plsc.load_gather(stage_b, [col_v[e, :]])
            trans_v[b, e, :] = v

    _fill(chunk0, 0)

    def _pair(p, carry):
        for b in range(2):
            k = p * 2 + b
            _wait_fill(b)

            @pl.when(k + 1 < CPT)
            def _():
                _fill(chunk0 + k + 1, 1 - b)

            @pl.when(k >= 2)
            def _():
                _wait_scat(b)

            _transpose(b)
            pltpu.async_copy(
                trans_v.at[b], acc.at[idx_v.at[k]], sem_s[b], add=True)
        return carry

    lax.fori_loop(0, CPT // 2, _pair, 0)
    _wait_scat(0)
    _wait_scat(1)

    @pl.when(wid < LEFT)
    def _():
        _fill(NW * CPT + wid, 0)
        _wait_fill(0)
        _transpose(0)
        pltpu.async_copy(
            trans_v.at[0], acc.at[idx_v.at[CPT]], sem_s[0], add=True)
        _wait_scat(0)

    plsc.subcore_barrier()

    pltpu.sync_copy(
        acc.at[pl.ds(row_base, ROWS_PER_TILE)],
        out_hbm.at[c].at[pl.ds(row_base, ROWS_PER_TILE)],
    )


def _combine_body(p_ref, o_ref):
    o_ref[...] = p_ref[0, : N_V * D // 128] + p_ref[1, : N_V * D // 128]


_combine = pl.pallas_call(
    _combine_body,
    out_shape=jax.ShapeDtypeStruct((N_V * D // 128, 128), jnp.float32),
)


def kernel(vertex_attr, edgeij_pair, edge_attr, g, batch):
    idx = (edgeij_pair.astype(jnp.int32)
           .reshape(2, N_CHUNKS, CHUNK).transpose(1, 0, 2))
    edges = (edge_attr.T.reshape(2, 8, N_CHUNKS, CHUNK)
             .transpose(0, 2, 1, 3).reshape(2, N_CHUNKS, 8 * CHUNK))
    col_table = jnp.asarray(_COL_TABLE)
    partials = _scatter_sc(idx, edges, col_table)
    summed = _combine(partials.reshape(NC, N_V_PAD * D // 128, 128))
    return summed.reshape(N_V, D)


# speedup vs baseline: 2.8997x; 2.8997x over previous
import functools

import jax
import jax.numpy as jnp
from jax import lax
from jax.experimental import pallas as pl
from jax.experimental.pallas import tpu as pltpu, tpu_sc as plsc

N_V = 10000
N_V_PAD = 10240
E = 320000
D = 16
CHUNK = 128
N_CHUNKS = E // CHUNK
NC, NS = 2, 16
EH_ROWS = N_CHUNKS // 2
BLK = 125
NBLK = EH_ROWS // BLK
GROUPS = BLK * CHUNK // D

_mesh = plsc.VectorSubcoreMesh(core_axis_name="c", subcore_axis_name="s")


@functools.partial(
    pl.kernel,
    out_type=jax.ShapeDtypeStruct((D, N_V), jnp.float32),
    mesh=_mesh,
    compiler_params=pltpu.CompilerParams(
        use_tc_tiling_on_sc=False, needs_layout_passes=False),
    scratch_types=[
        pltpu.VMEM((2, BLK, CHUNK), jnp.int32),
        pltpu.VMEM((2, BLK, CHUNK), jnp.float32),
        pltpu.VMEM((N_V_PAD,), jnp.float32),
        pltpu.VMEM((N_V_PAD,), jnp.float32),
        pltpu.VMEM_SHARED((NS, N_V_PAD), jnp.float32),
        pltpu.SemaphoreType.DMA,
        pltpu.SemaphoreType.DMA,
    ],
)
def _scatter_sc(idx_hbm, edge_hbm, out_hbm, idx_v, val_v, acc_v, peer_v,
                xch, sem_f0, sem_f1):
    c = lax.axis_index("c")
    s = lax.axis_index("s")
    f = s % 8
    h = s // 8
    row0 = h * EH_ROWS
    sem_f = (sem_f0, sem_f1)

    zero = jnp.zeros((D,), jnp.float32)

    @plsc.parallel_loop(0, N_V_PAD // D, unroll=8)
    def _z(i):
        acc_v[pl.ds(i * D, D)] = zero

    def _fill(blk, b):
        r = row0 + blk * BLK
        pltpu.async_copy(idx_hbm.at[pl.ds(r, BLK), 1, :], idx_v.at[b],
                         sem_f[b])
        pltpu.async_copy(edge_hbm.at[c, pl.ds(r, BLK), f, :], val_v.at[b],
                         sem_f[b])

    def _wait_fill(b):
        pltpu.make_async_copy(idx_hbm.at[pl.ds(0, BLK), 1, :],
                              idx_v.at[b], sem_f[b]).wait()
        pltpu.make_async_copy(edge_hbm.at[0, pl.ds(0, BLK), 0, :],
                              val_v.at[b], sem_f[b]).wait()

    _fill(0, 0)

    def _pair(p, carry):
        for b in range(2):
            k = p * 2 + b
            _wait_fill(b)

            @pl.when(k + 1 < NBLK)
            def _():
                _fill(k + 1, 1 - b)

            @plsc.parallel_loop(0, BLK, unroll=2)
            def _row(r):
                for j in range(CHUNK // D):
                    sl = pl.ds(j * D, D)
                    plsc.addupdate_scatter(
                        acc_v, [idx_v[b, r, sl]], val_v[b, r, sl])

        return carry

    lax.fori_loop(0, NBLK // 2, _pair, 0)

    pltpu.sync_copy(acc_v, xch.at[s])
    plsc.subcore_barrier()

    @pl.when(h == 0)
    def _():
        pltpu.sync_copy(xch.at[s + 8], peer_v)

        @plsc.parallel_loop(0, N_V_PAD // D, unroll=8)
        def _add(i):
            sl = pl.ds(i * D, D)
            acc_v[sl] = acc_v[sl] + peer_v[sl]

        pltpu.sync_copy(acc_v.at[pl.ds(0, N_V)],
                        out_hbm.at[c * 8 + f])


def kernel(vertex_attr, edgeij_pair, edge_attr, g, batch):
    idx = (edgeij_pair.astype(jnp.int32)
           .reshape(2, N_CHUNKS, CHUNK).transpose(1, 0, 2))
    edges = (edge_attr.T.reshape(2, 8, N_CHUNKS, CHUNK)
             .transpose(0, 2, 1, 3))
    out_t = _scatter_sc(idx, edges)
    return out_t.T


# speedup vs baseline: 2.9580x; 1.0201x over previous
import functools

import jax
import jax.numpy as jnp
from jax import lax
from jax.experimental import pallas as pl
from jax.experimental.pallas import tpu as pltpu, tpu_sc as plsc

N_V = 10000
N_V_PAD = 10240
E = 320000
D = 16
CHUNK = 128
N_CHUNKS = E // CHUNK
NC, NS = 2, 16
EH_ROWS = N_CHUNKS // 2
BLK = 125
NBLK = EH_ROWS // BLK
GROUPS = BLK * CHUNK // D

_mesh = plsc.VectorSubcoreMesh(core_axis_name="c", subcore_axis_name="s")


@functools.partial(
    pl.kernel,
    out_type=jax.ShapeDtypeStruct((D, N_V), jnp.float32),
    mesh=_mesh,
    compiler_params=pltpu.CompilerParams(
        use_tc_tiling_on_sc=False, needs_layout_passes=False),
    scratch_types=[
        pltpu.VMEM((2, BLK, CHUNK), jnp.int32),
        pltpu.VMEM((2, BLK, CHUNK), jnp.float32),
        pltpu.VMEM((N_V_PAD,), jnp.float32),
        pltpu.VMEM((N_V_PAD,), jnp.float32),
        pltpu.VMEM_SHARED((NS, N_V_PAD), jnp.float32),
        pltpu.VMEM_SHARED((N_CHUNKS, CHUNK), jnp.int32),
        pltpu.SemaphoreType.DMA,
        pltpu.SemaphoreType.DMA,
        pltpu.SemaphoreType.DMA,
        pltpu.SemaphoreType.DMA,
    ],
)
def _scatter_sc(idx_hbm, edge_hbm, out_hbm, idx_v, val_v, acc_v, peer_v,
                xch, idx_spm, sem_f0, sem_f1, sem_i0, sem_i1):
    c = lax.axis_index("c")
    s = lax.axis_index("s")
    f = s % 8
    h = s // 8
    row0 = h * EH_ROWS
    sem_f = (sem_f0, sem_f1)
    sem_i = (sem_i0, sem_i1)

    SROWS = N_CHUNKS // NS
    SLEFT = N_CHUNKS - SROWS * NS
    sr0 = pl.multiple_of(s * SROWS, 4)
    pltpu.sync_copy(idx_hbm.at[pl.ds(sr0, SROWS), 1, :],
                    idx_spm.at[pl.ds(sr0, SROWS)])

    @pl.when(s < SLEFT)
    def _():
        pltpu.sync_copy(idx_hbm.at[NS * SROWS + s, 1, :],
                        idx_spm.at[NS * SROWS + s])

    zero = jnp.zeros((D,), jnp.float32)

    @plsc.parallel_loop(0, N_V_PAD // D, unroll=8)
    def _z(i):
        acc_v[pl.ds(i * D, D)] = zero

    plsc.subcore_barrier()

    def _fill(blk, b):
        r = row0 + blk * BLK
        pltpu.async_copy(idx_spm.at[pl.ds(r, BLK)], idx_v.at[b],
                         sem_i[b])
        pltpu.async_copy(edge_hbm.at[c, pl.ds(r, BLK), f, :], val_v.at[b],
                         sem_f[b])

    def _wait_fill(b):
        pltpu.make_async_copy(idx_spm.at[pl.ds(0, BLK)],
                              idx_v.at[b], sem_i[b]).wait()
        pltpu.make_async_copy(edge_hbm.at[0, pl.ds(0, BLK), 0, :],
                              val_v.at[b], sem_f[b]).wait()

    _fill(0, 0)

    def _pair(p, carry):
        for b in range(2):
            k = p * 2 + b
            _wait_fill(b)

            @pl.when(k + 1 < NBLK)
            def _():
                _fill(k + 1, 1 - b)

            @plsc.parallel_loop(0, BLK, unroll=2)
            def _row(r):
                for j in range(CHUNK // D):
                    sl = pl.ds(j * D, D)
                    plsc.addupdate_scatter(
                        acc_v, [idx_v[b, r, sl]], val_v[b, r, sl])

        return carry

    lax.fori_loop(0, NBLK // 2, _pair, 0)

    pltpu.sync_copy(acc_v, xch.at[s])
    plsc.subcore_barrier()

    @pl.when(h == 0)
    def _():
        pltpu.sync_copy(xch.at[s + 8], peer_v)

        @plsc.parallel_loop(0, N_V_PAD // D, unroll=8)
        def _add(i):
            sl = pl.ds(i * D, D)
            acc_v[sl] = acc_v[sl] + peer_v[sl]

        pltpu.sync_copy(acc_v.at[pl.ds(0, N_V)],
                        out_hbm.at[c * 8 + f])


def kernel(vertex_attr, edgeij_pair, edge_attr, g, batch):
    idx = (edgeij_pair.astype(jnp.int32)
           .reshape(2, N_CHUNKS, CHUNK).transpose(1, 0, 2))
    edges = (edge_attr.T.reshape(2, 8, N_CHUNKS, CHUNK)
             .transpose(0, 2, 1, 3))
    out_t = _scatter_sc(idx, edges)
    return out_t.T
